# Initial kernel scaffold; baseline (speedup 1.0000x reference)
import jax
import jax.numpy as jnp
from jax.experimental import pallas as pl


def kernel(user_id, dow, hod, table):
    raise NotImplementedError("write your pallas kernel here")


# speedup vs baseline: 2.3440x; 2.3440x over previous
import functools

import jax
import jax.numpy as jnp
from jax import lax
from jax.experimental import pallas as pl
from jax.experimental.pallas import tpu as pltpu
from jax.experimental.pallas import tpu_sc as plsc

BATCH = 16384
EMB_D = 32
DOW_D = 7
HOD_D = 24
OH_D = DOW_D + HOD_D
OUT_D = EMB_D + OH_D
L = 16
NC, NS = 2, 16
NW = NC * NS
B_PER_W = BATCH // NW
GCHUNK = 128


def _sc_body(uid_hbm, dow_hbm, hod_hbm, tab_hbm, out_hbm,
             uid_v, dow_v, hod_v, idx_v, ebuf, buf, sem):
    wid = lax.axis_index("s") * NC + lax.axis_index("c")
    base = wid * B_PER_W
    vocab = tab_hbm.shape[0] - 1

    pltpu.sync_copy(uid_hbm.at[pl.ds(base, B_PER_W)], uid_v)
    pltpu.sync_copy(dow_hbm.at[pl.ds(base, B_PER_W)], dow_v)
    pltpu.sync_copy(hod_hbm.at[pl.ds(base, B_PER_W)], hod_v)

    for i in range(B_PER_W // L):
        u = uid_v[pl.ds(i * L, L)]
        ok = (u >= 0) & (u < vocab)
        idx_v[pl.ds(i * L, L)] = jnp.where(ok, u + 1, 0)

    copies = []
    for j in range(B_PER_W // GCHUNK):
        copies.append(pltpu.async_copy(
            tab_hbm.at[idx_v.at[pl.ds(j * GCHUNK, GCHUNK)]],
            ebuf.at[pl.ds(j * GCHUNK, GCHUNK)], sem))
    for cp in copies:
        cp.wait()

    zeros = jnp.zeros((L,), jnp.float32)

    def assemble(r, _):
        buf[r, pl.ds(0, L)] = ebuf[r, pl.ds(0, L)]
        buf[r, pl.ds(L, L)] = ebuf[r, pl.ds(L, L)]
        buf[r, pl.ds(EMB_D, L)] = zeros
        buf[r, pl.ds(OUT_D - L, L)] = zeros
        return _

    lax.fori_loop(0, B_PER_W, assemble, None)

    ones = jnp.full((L,), 1.0, jnp.float32)
    rows0 = lax.iota(jnp.int32, L)
    for i in range(B_PER_W // L):
        rows = rows0 + (i * L)
        d = dow_v[pl.ds(i * L, L)]
        h = hod_v[pl.ds(i * L, L)]
        plsc.store_scatter(buf, [rows, d + EMB_D], ones)
        plsc.store_scatter(buf, [rows, h + (EMB_D + DOW_D)], ones)

    pltpu.sync_copy(buf, out_hbm.at[pl.ds(base, B_PER_W)])


@functools.partial(
    pl.kernel,
    out_type=jax.ShapeDtypeStruct((BATCH, OUT_D), jnp.float32),
    mesh=plsc.VectorSubcoreMesh(core_axis_name="c", subcore_axis_name="s",
                                num_cores=NC, num_subcores=NS),
    scratch_types=[
        pltpu.VMEM((B_PER_W,), jnp.int32),
        pltpu.VMEM((B_PER_W,), jnp.int32),
        pltpu.VMEM((B_PER_W,), jnp.int32),
        pltpu.VMEM((B_PER_W,), jnp.int32),
        pltpu.VMEM((B_PER_W, EMB_D), jnp.float32),
        pltpu.VMEM((B_PER_W, OUT_D), jnp.float32),
        pltpu.SemaphoreType.DMA,
    ],
    compiler_params=pltpu.CompilerParams(use_tc_tiling_on_sc=False,
                                         needs_layout_passes=False),
)
def _sc_query_model(uid_hbm, dow_hbm, hod_hbm, tab_hbm, out_hbm,
                    uid_v, dow_v, hod_v, idx_v, ebuf, buf, sem):
    _sc_body(uid_hbm, dow_hbm, hod_hbm, tab_hbm, out_hbm,
             uid_v, dow_v, hod_v, idx_v, ebuf, buf, sem)


def kernel(user_id, dow, hod, table):
    return _sc_query_model(user_id, dow, hod, table)


# speedup vs baseline: 2.4502x; 1.0453x over previous
import functools

import jax
import jax.numpy as jnp
from jax import lax
from jax.experimental import pallas as pl
from jax.experimental.pallas import tpu as pltpu
from jax.experimental.pallas import tpu_sc as plsc

BATCH = 16384
EMB_D = 32
DOW_D = 7
HOD_D = 24
OH_D = DOW_D + HOD_D
OUT_D = EMB_D + OH_D
L = 16
NC, NS = 2, 16
NW = NC * NS
B_PER_W = BATCH // NW
GCHUNK = 128
NCH = B_PER_W // GCHUNK


def _sc_body(uid_hbm, dow_hbm, hod_hbm, tab_hbm, out_hbm,
             uid_v, dow_v, hod_v, idx_v, ebuf, buf, gsem, osem):
    wid = lax.axis_index("s") * NC + lax.axis_index("c")
    base = wid * B_PER_W
    vocab = tab_hbm.shape[0] - 1

    pltpu.sync_copy(uid_hbm.at[pl.ds(base, B_PER_W)], uid_v)
    pltpu.sync_copy(dow_hbm.at[pl.ds(base, B_PER_W)], dow_v)
    pltpu.sync_copy(hod_hbm.at[pl.ds(base, B_PER_W)], hod_v)

    gcopies = []
    for j in range(NCH):
        @plsc.parallel_loop(j * (GCHUNK // L), (j + 1) * (GCHUNK // L),
                            unroll=4)
        def _(i):
            u = uid_v[pl.ds(i * L, L)]
            ok = (u >= 0) & (u < vocab)
            idx_v[pl.ds(i * L, L)] = jnp.where(ok, u + 1, 0)

        gcopies.append(pltpu.async_copy(
            tab_hbm.at[idx_v.at[pl.ds(j * GCHUNK, GCHUNK)]],
            ebuf.at[pl.ds(j * GCHUNK, GCHUNK)], gsem.at[j]))

    zeros = jnp.zeros((L,), jnp.float32)
    ones = jnp.full((L,), 1.0, jnp.float32)
    rows0 = lax.iota(jnp.int32, L)
    ocopies = []
    for j in range(NCH):
        gcopies[j].wait()

        @plsc.parallel_loop(j * GCHUNK, (j + 1) * GCHUNK, unroll=4)
        def _(r):
            buf[r, pl.ds(0, L)] = ebuf[r, pl.ds(0, L)]
            buf[r, pl.ds(L, L)] = ebuf[r, pl.ds(L, L)]
            buf[r, pl.ds(EMB_D, L)] = zeros
            buf[r, pl.ds(OUT_D - L, L)] = zeros

        @plsc.parallel_loop(j * (GCHUNK // L), (j + 1) * (GCHUNK // L),
                            unroll=2)
        def _(i):
            rows = rows0 + i * L
            d = dow_v[pl.ds(i * L, L)]
            h = hod_v[pl.ds(i * L, L)]
            plsc.store_scatter(buf, [rows, d + EMB_D], ones)
            plsc.store_scatter(buf, [rows, h + (EMB_D + DOW_D)], ones)

        ocopies.append(pltpu.async_copy(
            buf.at[pl.ds(j * GCHUNK, GCHUNK)],
            out_hbm.at[pl.ds(base + j * GCHUNK, GCHUNK)], osem))
    for cp in ocopies:
        cp.wait()


@functools.partial(
    pl.kernel,
    out_type=jax.ShapeDtypeStruct((BATCH, OUT_D), jnp.float32),
    mesh=plsc.VectorSubcoreMesh(core_axis_name="c", subcore_axis_name="s",
                                num_cores=NC, num_subcores=NS),
    scratch_types=[
        pltpu.VMEM((B_PER_W,), jnp.int32),
        pltpu.VMEM((B_PER_W,), jnp.int32),
        pltpu.VMEM((B_PER_W,), jnp.int32),
        pltpu.VMEM((B_PER_W,), jnp.int32),
        pltpu.VMEM((B_PER_W, EMB_D), jnp.float32),
        pltpu.VMEM((B_PER_W, OUT_D), jnp.float32),
        pltpu.SemaphoreType.DMA((NCH,)),
        pltpu.SemaphoreType.DMA,
    ],
    compiler_params=pltpu.CompilerParams(use_tc_tiling_on_sc=False,
                                         needs_layout_passes=False),
)
def _sc_query_model(uid_hbm, dow_hbm, hod_hbm, tab_hbm, out_hbm,
                    uid_v, dow_v, hod_v, idx_v, ebuf, buf, gsem, osem):
    _sc_body(uid_hbm, dow_hbm, hod_hbm, tab_hbm, out_hbm,
             uid_v, dow_v, hod_v, idx_v, ebuf, buf, gsem, osem)


def kernel(user_id, dow, hod, table):
    return _sc_query_model(user_id, dow, hod, table)


# speedup vs baseline: 2.8216x; 1.1516x over previous
import functools

import jax
import jax.numpy as jnp
from jax import lax
from jax.experimental import pallas as pl
from jax.experimental.pallas import tpu as pltpu
from jax.experimental.pallas import tpu_sc as plsc

BATCH = 16384
EMB_D = 32
DOW_D = 7
HOD_D = 24
OH_D = DOW_D + HOD_D
OUT_D = EMB_D + OH_D
L = 16
NC, NS = 2, 16
NW = NC * NS
B_PER_W = BATCH // NW
GCHUNK = 128
NCH = B_PER_W // GCHUNK
PAD_D = 128


def _sc_body(uid_hbm, dow_hbm, hod_hbm, tab_hbm, out_hbm,
             uid_v, dow_v, hod_v, idx_v, ebuf, buf, gsem, osem):
    wid = lax.axis_index("s") * NC + lax.axis_index("c")
    base = wid * B_PER_W
    vocab = tab_hbm.shape[0] - 1

    pltpu.sync_copy(uid_hbm.at[pl.ds(base, B_PER_W)], uid_v)
    pltpu.sync_copy(dow_hbm.at[pl.ds(base, B_PER_W)], dow_v)
    pltpu.sync_copy(hod_hbm.at[pl.ds(base, B_PER_W)], hod_v)

    gcopies = []
    for j in range(NCH):
        @plsc.parallel_loop(j * (GCHUNK // L), (j + 1) * (GCHUNK // L),
                            unroll=4)
        def _(i):
            u = uid_v[pl.ds(i * L, L)]
            ok = (u >= 0) & (u < vocab)
            idx_v[pl.ds(i * L, L)] = jnp.where(ok, u + 1, 0)

        gcopies.append(pltpu.async_copy(
            tab_hbm.at[idx_v.at[pl.ds(j * GCHUNK, GCHUNK)]],
            ebuf.at[pl.ds(j * GCHUNK, GCHUNK)], gsem.at[j]))

    zeros = jnp.zeros((L,), jnp.float32)
    ones = jnp.full((L,), 1.0, jnp.float32)
    rows0 = lax.iota(jnp.int32, L)
    ocopies = []
    for j in range(NCH):
        gcopies[j].wait()

        @plsc.parallel_loop(j * GCHUNK, (j + 1) * GCHUNK, unroll=4)
        def _(r):
            buf[r, pl.ds(0, L)] = ebuf[r, pl.ds(0, L)]
            buf[r, pl.ds(L, L)] = ebuf[r, pl.ds(L, L)]
            buf[r, pl.ds(EMB_D, L)] = zeros
            buf[r, pl.ds(OUT_D - L, L)] = zeros

        @plsc.parallel_loop(j * (GCHUNK // L), (j + 1) * (GCHUNK // L),
                            unroll=2)
        def _(i):
            rows = rows0 + i * L
            d = dow_v[pl.ds(i * L, L)]
            h = hod_v[pl.ds(i * L, L)]
            plsc.store_scatter(buf, [rows, d + EMB_D], ones)
            plsc.store_scatter(buf, [rows, h + (EMB_D + DOW_D)], ones)

        ocopies.append(pltpu.async_copy(
            buf.at[pl.ds(j * GCHUNK, GCHUNK)],
            out_hbm.at[pl.ds(base + j * GCHUNK, GCHUNK)], osem))
    for cp in ocopies:
        cp.wait()


@functools.partial(
    pl.kernel,
    out_type=jax.ShapeDtypeStruct((BATCH, PAD_D), jnp.float32),
    mesh=plsc.VectorSubcoreMesh(core_axis_name="c", subcore_axis_name="s",
                                num_cores=NC, num_subcores=NS),
    scratch_types=[
        pltpu.VMEM((B_PER_W,), jnp.int32),
        pltpu.VMEM((B_PER_W,), jnp.int32),
        pltpu.VMEM((B_PER_W,), jnp.int32),
        pltpu.VMEM((B_PER_W,), jnp.int32),
        pltpu.VMEM((B_PER_W, EMB_D), jnp.float32),
        pltpu.VMEM((B_PER_W, PAD_D), jnp.float32),
        pltpu.SemaphoreType.DMA((NCH,)),
        pltpu.SemaphoreType.DMA,
    ],
    compiler_params=pltpu.CompilerParams(use_tc_tiling_on_sc=False,
                                         needs_layout_passes=False),
)
def _sc_query_model(uid_hbm, dow_hbm, hod_hbm, tab_hbm, out_hbm,
                    uid_v, dow_v, hod_v, idx_v, ebuf, buf, gsem, osem):
    _sc_body(uid_hbm, dow_hbm, hod_hbm, tab_hbm, out_hbm,
             uid_v, dow_v, hod_v, idx_v, ebuf, buf, gsem, osem)


def kernel(user_id, dow, hod, table):
    padded = _sc_query_model(user_id, dow, hod, table)
    return lax.slice(padded, (0, 0), (BATCH, OUT_D))


# speedup vs baseline: 2.9683x; 1.0520x over previous
import functools

import jax
import jax.numpy as jnp
from jax import lax
from jax.experimental import pallas as pl
from jax.experimental.pallas import tpu as pltpu
from jax.experimental.pallas import tpu_sc as plsc

BATCH = 16384
EMB_D = 32
DOW_D = 7
HOD_D = 24
OH_D = DOW_D + HOD_D
OUT_D = EMB_D + OH_D
L = 16
NC, NS = 2, 16
NW = NC * NS
B_PER_W = BATCH // NW
GCHUNK = 128
NCH = B_PER_W // GCHUNK
PAD_D = 128


def _sc_body(uid_hbm, dow_hbm, hod_hbm, tab_hbm, out_hbm,
             uid_v, dow_v, hod_v, idx_v, ebuf, buf, gsem, osem):
    wid = lax.axis_index("s") * NC + lax.axis_index("c")
    base = wid * B_PER_W
    vocab = tab_hbm.shape[0] - 1

    pltpu.sync_copy(uid_hbm.at[pl.ds(base, B_PER_W)], uid_v)
    pltpu.sync_copy(dow_hbm.at[pl.ds(base, B_PER_W)], dow_v)
    pltpu.sync_copy(hod_hbm.at[pl.ds(base, B_PER_W)], hod_v)

    gcopies = []
    for j in range(NCH):
        @plsc.parallel_loop(j * (GCHUNK // L), (j + 1) * (GCHUNK // L),
                            unroll=4)
        def _(i):
            u = uid_v[pl.ds(i * L, L)]
            ok = (u >= 0) & (u < vocab)
            idx_v[pl.ds(i * L, L)] = jnp.where(ok, u + 1, 0)

        gcopies.append(pltpu.async_copy(
            tab_hbm.at[idx_v.at[pl.ds(j * GCHUNK, GCHUNK)]],
            ebuf.at[pl.ds(j * GCHUNK, GCHUNK)], gsem.at[j]))

    zeros = jnp.zeros((L,), jnp.float32)
    ones = jnp.full((L,), 1.0, jnp.float32)
    rows0 = lax.iota(jnp.int32, L)
    ocopies = []
    for j in range(NCH):
        gcopies[j].wait()

        @plsc.parallel_loop(j * GCHUNK, (j + 1) * GCHUNK, unroll=4)
        def _(r):
            buf[r, pl.ds(0, L)] = ebuf[r, pl.ds(0, L)]
            buf[r, pl.ds(L, L)] = ebuf[r, pl.ds(L, L)]
            buf[r, pl.ds(EMB_D, L)] = zeros
            buf[r, pl.ds(3 * L, L)] = zeros

        @plsc.parallel_loop(j * (GCHUNK // L), (j + 1) * (GCHUNK // L),
                            unroll=2)
        def _(i):
            rows = rows0 + i * L
            d = dow_v[pl.ds(i * L, L)]
            h = hod_v[pl.ds(i * L, L)]
            plsc.store_scatter(buf, [rows, d + EMB_D], ones)
            plsc.store_scatter(buf, [rows, h + (EMB_D + DOW_D)], ones)

        ocopies.append(pltpu.async_copy(
            buf.at[pl.ds(j * GCHUNK, GCHUNK)],
            out_hbm.at[pl.ds(base + j * GCHUNK, GCHUNK), pl.ds(0, 2 * EMB_D)],
            osem))
    for cp in ocopies:
        cp.wait()


@functools.partial(
    pl.kernel,
    out_type=jax.ShapeDtypeStruct((BATCH, PAD_D), jnp.float32),
    mesh=plsc.VectorSubcoreMesh(core_axis_name="c", subcore_axis_name="s",
                                num_cores=NC, num_subcores=NS),
    scratch_types=[
        pltpu.VMEM((B_PER_W,), jnp.int32),
        pltpu.VMEM((B_PER_W,), jnp.int32),
        pltpu.VMEM((B_PER_W,), jnp.int32),
        pltpu.VMEM((B_PER_W,), jnp.int32),
        pltpu.VMEM((B_PER_W, EMB_D), jnp.float32),
        pltpu.VMEM((B_PER_W, 2 * EMB_D), jnp.float32),
        pltpu.SemaphoreType.DMA((NCH,)),
        pltpu.SemaphoreType.DMA,
    ],
    compiler_params=pltpu.CompilerParams(use_tc_tiling_on_sc=False,
                                         needs_layout_passes=False),
)
def _sc_query_model(uid_hbm, dow_hbm, hod_hbm, tab_hbm, out_hbm,
                    uid_v, dow_v, hod_v, idx_v, ebuf, buf, gsem, osem):
    _sc_body(uid_hbm, dow_hbm, hod_hbm, tab_hbm, out_hbm,
             uid_v, dow_v, hod_v, idx_v, ebuf, buf, gsem, osem)


def kernel(user_id, dow, hod, table):
    padded = _sc_query_model(user_id, dow, hod, table)
    return lax.slice(padded, (0, 0), (BATCH, OUT_D))
